# baseline (device time: 19296 ns/iter reference)
import jax
import jax.numpy as jnp
from jax import lax
from jax.experimental import pallas as pl
from jax.experimental.pallas import tpu as pltpu

N_DEV = 4


def kernel(x, W1, W2):
    m, k = x.shape
    kh, h_per = W1.shape
    n = W2.shape[1]
    mq = m // N_DEV

    def body(x_hbm, w1_hbm, w2_hbm, out_hbm,
             xv, w1v, w2v, outv,
             part_buf, rs_buf, ag_src,
             in_sems, out_sem,
             rs_send, rs_recv, ag_send, ag_recv):
        my = lax.axis_index("i")

        cp_x = pltpu.make_async_copy(x_hbm, xv, in_sems.at[0])
        cp_w1 = pltpu.make_async_copy(w1_hbm, w1v, in_sems.at[1])
        cp_w2 = pltpu.make_async_copy(w2_hbm, w2v, in_sems.at[2])
        cp_x.start()
        cp_w1.start()
        cp_w2.start()

        barrier_sem = pltpu.get_barrier_semaphore()
        for t in range(N_DEV - 1):
            pl.semaphore_signal(
                barrier_sem, inc=1,
                device_id=(lax.rem(my + 1 + t, N_DEV),),
                device_id_type=pl.DeviceIdType.MESH,
            )

        cp_x.wait()
        cp_w1.wait()
        cp_w2.wait()
        w1b = w1v[...].astype(jnp.bfloat16)
        w2b = w2v[...].astype(jnp.bfloat16)

        def quarter(dst):
            xc = xv[pl.ds(dst * mq, mq), :].astype(jnp.bfloat16)
            hc = jnp.dot(xc, w1b, preferred_element_type=jnp.float32)
            hc = jnp.maximum(hc, 0.0).astype(jnp.bfloat16)
            return jnp.dot(hc, w2b, preferred_element_type=jnp.float32)

        pl.semaphore_wait(barrier_sem, N_DEV - 1)

        rs = []
        for t in range(N_DEV - 1):
            dst = lax.rem(my + 1 + t, N_DEV)
            part_buf[t] = quarter(dst).astype(jnp.bfloat16)
            rdma = pltpu.make_async_remote_copy(
                src_ref=part_buf.at[t],
                dst_ref=rs_buf.at[2 - t],
                send_sem=rs_send.at[t],
                recv_sem=rs_recv.at[2 - t],
                device_id=(dst,),
                device_id_type=pl.DeviceIdType.MESH,
            )
            rdma.start()
            rs.append(rdma)

        red = quarter(my)
        for rdma in rs:
            rdma.wait()
        for s in range(N_DEV - 1):
            red = red + rs_buf[s].astype(jnp.float32)
        redb = red.astype(jnp.bfloat16)
        outv[pl.ds(my * mq, mq), :] = redb
        ag_src[...] = redb

        ag = []
        for t in range(N_DEV - 1):
            dst = lax.rem(my + 1 + t, N_DEV)
            rdma = pltpu.make_async_remote_copy(
                src_ref=ag_src,
                dst_ref=outv.at[pl.ds(my * mq, mq), :],
                send_sem=ag_send.at[t],
                recv_sem=ag_recv.at[2 - t],
                device_id=(dst,),
                device_id_type=pl.DeviceIdType.MESH,
            )
            rdma.start()
            ag.append(rdma)
        for rdma in ag:
            rdma.wait()

        cp_out = pltpu.make_async_copy(outv, out_hbm, out_sem)
        cp_out.start()
        cp_out.wait()

    return pl.pallas_call(
        body,
        out_shape=jax.ShapeDtypeStruct((m, n), jnp.bfloat16),
        in_specs=[
            pl.BlockSpec(memory_space=pl.ANY),
            pl.BlockSpec(memory_space=pl.ANY),
            pl.BlockSpec(memory_space=pl.ANY),
        ],
        out_specs=pl.BlockSpec(memory_space=pl.ANY),
        scratch_shapes=[
            pltpu.VMEM((m, k), jnp.float32),
            pltpu.VMEM((kh, h_per), jnp.float32),
            pltpu.VMEM((h_per, n), jnp.float32),
            pltpu.VMEM((m, n), jnp.bfloat16),
            pltpu.VMEM((N_DEV - 1, mq, n), jnp.bfloat16),
            pltpu.VMEM((N_DEV - 1, mq, n), jnp.bfloat16),
            pltpu.VMEM((mq, n), jnp.bfloat16),
            pltpu.SemaphoreType.DMA((3,)),
            pltpu.SemaphoreType.DMA,
            pltpu.SemaphoreType.DMA((N_DEV - 1,)),
            pltpu.SemaphoreType.DMA((N_DEV - 1,)),
            pltpu.SemaphoreType.DMA((N_DEV - 1,)),
            pltpu.SemaphoreType.DMA((N_DEV - 1,)),
        ],
        compiler_params=pltpu.CompilerParams(collective_id=0),
    )(x, W1, W2)


# device time: 18695 ns/iter; 1.0321x vs baseline; 1.0321x over previous
import jax
import jax.numpy as jnp
from jax import lax
from jax.experimental import pallas as pl
from jax.experimental.pallas import tpu as pltpu

N_DEV = 4


def kernel(x, W1, W2):
    m, k = x.shape
    kh, h_per = W1.shape
    n = W2.shape[1]
    mq = m // N_DEV

    def body(x_hbm, w1_hbm, w2_hbm, out_hbm,
             xv, w1v, w2v, hv, outv,
             part_buf, rs_buf, ag_src,
             in_sems, out_sems,
             rs_send, rs_recv, ag_send, ag_recv):
        my = lax.axis_index("i")

        cp_x = pltpu.make_async_copy(x_hbm, xv, in_sems.at[0])
        cp_w1 = pltpu.make_async_copy(w1_hbm, w1v, in_sems.at[1])
        cp_w2 = pltpu.make_async_copy(w2_hbm, w2v, in_sems.at[2])
        cp_x.start()
        cp_w1.start()
        cp_w2.start()

        barrier_sem = pltpu.get_barrier_semaphore()
        for t in range(N_DEV - 1):
            pl.semaphore_signal(
                barrier_sem, inc=1,
                device_id=(lax.rem(my + 1 + t, N_DEV),),
                device_id_type=pl.DeviceIdType.MESH,
            )

        cp_x.wait()
        cp_w1.wait()
        xb = xv[...].astype(jnp.bfloat16)
        w1b = w1v[...].astype(jnp.bfloat16)
        h = jnp.dot(xb, w1b, preferred_element_type=jnp.float32)
        hv[...] = jnp.maximum(h, 0.0).astype(jnp.bfloat16)

        cp_w2.wait()
        w2b = w2v[...].astype(jnp.bfloat16)

        def quarter(dst):
            hc = hv[pl.ds(dst * mq, mq), :]
            return jnp.dot(hc, w2b, preferred_element_type=jnp.float32)

        pl.semaphore_wait(barrier_sem, N_DEV - 1)

        rs = []
        for t in range(N_DEV - 1):
            dst = lax.rem(my + 1 + t, N_DEV)
            part_buf[t] = quarter(dst).astype(jnp.bfloat16)
            rdma = pltpu.make_async_remote_copy(
                src_ref=part_buf.at[t],
                dst_ref=rs_buf.at[2 - t],
                send_sem=rs_send.at[t],
                recv_sem=rs_recv.at[2 - t],
                device_id=(dst,),
                device_id_type=pl.DeviceIdType.MESH,
            )
            rdma.start()
            rs.append(rdma)

        red = quarter(my)
        for rdma in rs:
            rdma.wait()
        for s in range(N_DEV - 1):
            red = red + rs_buf[s].astype(jnp.float32)
        redb = red.astype(jnp.bfloat16)
        outv[pl.ds(my * mq, mq), :] = redb
        ag_src[...] = redb

        my_rows_v = outv.at[pl.ds(my * mq, mq), :]
        cp_mine = pltpu.make_async_copy(
            my_rows_v, out_hbm.at[pl.ds(my * mq, mq), :], out_sems.at[N_DEV - 1])
        cp_mine.start()

        ag = []
        for t in range(N_DEV - 1):
            dst = lax.rem(my + 1 + t, N_DEV)
            rdma = pltpu.make_async_remote_copy(
                src_ref=ag_src,
                dst_ref=my_rows_v,
                send_sem=ag_send.at[t],
                recv_sem=ag_recv.at[2 - t],
                device_id=(dst,),
                device_id_type=pl.DeviceIdType.MESH,
            )
            rdma.start()
            ag.append(rdma)

        out_cps = [cp_mine]
        for t in range(N_DEV - 1):
            ag[t].wait_send()
        for t in range(N_DEV - 1):
            ag[t].wait_recv()
            src_dev = lax.rem(my + 3 - t, N_DEV)
            rows = pl.ds(src_dev * mq, mq)
            cp = pltpu.make_async_copy(
                outv.at[rows, :], out_hbm.at[rows, :], out_sems.at[2 - t])
            cp.start()
            out_cps.append(cp)
        for cp in out_cps:
            cp.wait()

    return pl.pallas_call(
        body,
        out_shape=jax.ShapeDtypeStruct((m, n), jnp.bfloat16),
        in_specs=[
            pl.BlockSpec(memory_space=pl.ANY),
            pl.BlockSpec(memory_space=pl.ANY),
            pl.BlockSpec(memory_space=pl.ANY),
        ],
        out_specs=pl.BlockSpec(memory_space=pl.ANY),
        scratch_shapes=[
            pltpu.VMEM((m, k), jnp.float32),
            pltpu.VMEM((kh, h_per), jnp.float32),
            pltpu.VMEM((h_per, n), jnp.float32),
            pltpu.VMEM((m, h_per), jnp.bfloat16),
            pltpu.VMEM((m, n), jnp.bfloat16),
            pltpu.VMEM((N_DEV - 1, mq, n), jnp.bfloat16),
            pltpu.VMEM((N_DEV - 1, mq, n), jnp.bfloat16),
            pltpu.VMEM((mq, n), jnp.bfloat16),
            pltpu.SemaphoreType.DMA((3,)),
            pltpu.SemaphoreType.DMA((N_DEV,)),
            pltpu.SemaphoreType.DMA((N_DEV - 1,)),
            pltpu.SemaphoreType.DMA((N_DEV - 1,)),
            pltpu.SemaphoreType.DMA((N_DEV - 1,)),
            pltpu.SemaphoreType.DMA((N_DEV - 1,)),
        ],
        compiler_params=pltpu.CompilerParams(collective_id=0),
    )(x, W1, W2)


# device time: 18125 ns/iter; 1.0646x vs baseline; 1.0314x over previous
import jax
import jax.numpy as jnp
from jax import lax
from jax.experimental import pallas as pl
from jax.experimental.pallas import tpu as pltpu

N_DEV = 4


def kernel(x, W1, W2):
    m, k = x.shape
    h_per, n = W2.shape
    mq = m // N_DEV

    def body(x_ref, w1_ref, w2_hbm, out_ref,
             w2v, hv, part_buf, rs_buf, ag_src,
             w2_sem, rs_send, rs_recv, ag_send, ag_recv):
        my = lax.axis_index("i")

        cp_w2 = pltpu.make_async_copy(w2_hbm, w2v, w2_sem)
        cp_w2.start()

        barrier_sem = pltpu.get_barrier_semaphore()
        for t in range(N_DEV - 1):
            pl.semaphore_signal(
                barrier_sem, inc=1,
                device_id=(lax.rem(my + 1 + t, N_DEV),),
                device_id_type=pl.DeviceIdType.MESH,
            )

        xb = x_ref[...].astype(jnp.bfloat16)
        w1b = w1_ref[...].astype(jnp.bfloat16)
        h = jnp.dot(xb, w1b, preferred_element_type=jnp.float32)
        hv[...] = jnp.maximum(h, 0.0).astype(jnp.bfloat16)

        cp_w2.wait()
        w2b = w2v[...].astype(jnp.bfloat16)

        def quarter(dst):
            hc = hv[pl.ds(dst * mq, mq), :]
            return jnp.dot(hc, w2b, preferred_element_type=jnp.float32)

        pl.semaphore_wait(barrier_sem, N_DEV - 1)

        rs = []
        for t in range(N_DEV - 1):
            dst = lax.rem(my + 1 + t, N_DEV)
            part_buf[t] = quarter(dst).astype(jnp.bfloat16)
            rdma = pltpu.make_async_remote_copy(
                src_ref=part_buf.at[t],
                dst_ref=rs_buf.at[2 - t],
                send_sem=rs_send.at[t],
                recv_sem=rs_recv.at[2 - t],
                device_id=(dst,),
                device_id_type=pl.DeviceIdType.MESH,
            )
            rdma.start()
            rs.append(rdma)

        red = quarter(my)
        for rdma in rs:
            rdma.wait()
        for s in range(N_DEV - 1):
            red = red + rs_buf[s].astype(jnp.float32)
        redb = red.astype(jnp.bfloat16)
        out_ref[pl.ds(my * mq, mq), :] = redb
        ag_src[...] = redb

        ag = []
        for t in range(N_DEV - 1):
            dst = lax.rem(my + 1 + t, N_DEV)
            rdma = pltpu.make_async_remote_copy(
                src_ref=ag_src,
                dst_ref=out_ref.at[pl.ds(my * mq, mq), :],
                send_sem=ag_send.at[t],
                recv_sem=ag_recv.at[2 - t],
                device_id=(dst,),
                device_id_type=pl.DeviceIdType.MESH,
            )
            rdma.start()
            ag.append(rdma)
        for rdma in ag:
            rdma.wait()

    return pl.pallas_call(
        body,
        out_shape=jax.ShapeDtypeStruct((m, n), jnp.bfloat16),
        in_specs=[
            pl.BlockSpec(memory_space=pltpu.VMEM),
            pl.BlockSpec(memory_space=pltpu.VMEM),
            pl.BlockSpec(memory_space=pl.ANY),
        ],
        out_specs=pl.BlockSpec(memory_space=pltpu.VMEM),
        scratch_shapes=[
            pltpu.VMEM((h_per, n), jnp.float32),
            pltpu.VMEM((m, h_per), jnp.bfloat16),
            pltpu.VMEM((N_DEV - 1, mq, n), jnp.bfloat16),
            pltpu.VMEM((N_DEV - 1, mq, n), jnp.bfloat16),
            pltpu.VMEM((mq, n), jnp.bfloat16),
            pltpu.SemaphoreType.DMA,
            pltpu.SemaphoreType.DMA((N_DEV - 1,)),
            pltpu.SemaphoreType.DMA((N_DEV - 1,)),
            pltpu.SemaphoreType.DMA((N_DEV - 1,)),
            pltpu.SemaphoreType.DMA((N_DEV - 1,)),
        ],
        compiler_params=pltpu.CompilerParams(collective_id=0),
    )(x, W1, W2)


# device time: 16043 ns/iter; 1.2028x vs baseline; 1.1298x over previous
import jax
import jax.numpy as jnp
from jax import lax
from jax.experimental import pallas as pl
from jax.experimental.pallas import tpu as pltpu

N_DEV = 4
NSPLIT = 2


def kernel(x, W1, W2):
    m, k = x.shape
    n = W2.shape[1]
    mq = m // N_DEV
    rh = mq // NSPLIT

    def body(x_ref, w1_ref, w2_ref, out_ref,
             hv, part_buf, rs_buf, ag_src,
             rs_send, rs_recv, ag_send, ag_recv):
        my = lax.axis_index("i")

        barrier_sem = pltpu.get_barrier_semaphore()
        for t in range(N_DEV - 1):
            pl.semaphore_signal(
                barrier_sem, inc=1,
                device_id=(lax.rem(my + 1 + t, N_DEV),),
                device_id_type=pl.DeviceIdType.MESH,
            )

        h = jnp.dot(x_ref[...], w1_ref[...],
                    preferred_element_type=jnp.float32)
        hv[...] = jnp.maximum(h, 0.0).astype(jnp.bfloat16)
        w2b = w2_ref[...]

        def quarter(dst):
            hc = hv[pl.ds(dst * mq, mq), :]
            return jnp.dot(hc, w2b, preferred_element_type=jnp.float32)

        pl.semaphore_wait(barrier_sem, N_DEV - 1)

        def rs_send_blk(t, dst, blk):
            rdma = pltpu.make_async_remote_copy(
                src_ref=part_buf.at[t, pl.ds(blk * rh, rh), :],
                dst_ref=rs_buf.at[blk, 2 - t],
                send_sem=rs_send.at[3 * blk + t],
                recv_sem=rs_recv.at[3 * blk + (2 - t)],
                device_id=(dst,),
                device_id_type=pl.DeviceIdType.MESH,
            )
            rdma.start()
            return rdma

        rs = [[] for _ in range(NSPLIT)]
        for t in range(N_DEV - 1):
            dst = lax.rem(my + 1 + t, N_DEV)
            part_buf[t] = quarter(dst).astype(jnp.bfloat16)
            rs[0].append(rs_send_blk(t, dst, 0))
        for blk in range(1, NSPLIT):
            for t in range(N_DEV - 1):
                dst = lax.rem(my + 1 + t, N_DEV)
                rs[blk].append(rs_send_blk(t, dst, blk))
        red = quarter(my)

        def ag_send_blk(t, dst, blk):
            rdma = pltpu.make_async_remote_copy(
                src_ref=ag_src.at[blk],
                dst_ref=out_ref.at[pl.ds(my * mq + blk * rh, rh), :],
                send_sem=ag_send.at[3 * blk + t],
                recv_sem=ag_recv.at[3 * blk + (2 - t)],
                device_id=(dst,),
                device_id_type=pl.DeviceIdType.MESH,
            )
            rdma.start()
            return rdma

        ag = []
        for blk in range(NSPLIT):
            for rdma in rs[blk]:
                rdma.wait()
            redh = red[blk * rh:(blk + 1) * rh, :]
            for s in range(N_DEV - 1):
                redh = redh + rs_buf[blk, s].astype(jnp.float32)
            redhb = redh.astype(jnp.bfloat16)
            out_ref[pl.ds(my * mq + blk * rh, rh), :] = redhb
            ag_src[blk] = redhb
            for t in range(N_DEV - 1):
                dst = lax.rem(my + 1 + t, N_DEV)
                ag.append(ag_send_blk(t, dst, blk))
        for rdma in ag:
            rdma.wait()

    inner = pl.pallas_call(
        body,
        out_shape=jax.ShapeDtypeStruct((m, n), jnp.bfloat16),
        in_specs=[
            pl.BlockSpec(memory_space=pltpu.VMEM),
            pl.BlockSpec(memory_space=pltpu.VMEM),
            pl.BlockSpec(memory_space=pltpu.VMEM),
        ],
        out_specs=pl.BlockSpec(memory_space=pltpu.VMEM),
        scratch_shapes=[
            pltpu.VMEM((m, W1.shape[1]), jnp.bfloat16),
            pltpu.VMEM((N_DEV - 1, mq, n), jnp.bfloat16),
            pltpu.VMEM((NSPLIT, N_DEV - 1, rh, n), jnp.bfloat16),
            pltpu.VMEM((NSPLIT, rh, n), jnp.bfloat16),
            pltpu.SemaphoreType.DMA((3 * NSPLIT,)),
            pltpu.SemaphoreType.DMA((3 * NSPLIT,)),
            pltpu.SemaphoreType.DMA((3 * NSPLIT,)),
            pltpu.SemaphoreType.DMA((3 * NSPLIT,)),
        ],
        compiler_params=pltpu.CompilerParams(collective_id=0),
    )
    return inner(
        x.astype(jnp.bfloat16),
        W1.astype(jnp.bfloat16),
        W2.astype(jnp.bfloat16),
    )
